# baseline (speedup 1.0000x reference)
import functools

import jax
import jax.numpy as jnp
from jax import lax
from jax.experimental import pallas as pl
from jax.experimental.pallas import tpu as pltpu
from jax.experimental.pallas import tpu_sc as plsc

_B = 16384
_D = 128
_K0 = 100
_K1 = 10
_KP = 128
_CHUNK = 128
_NC = 2
_NS = 16
_NW = _NC * _NS
_NSPLIT = 4
_HB = _B // _NSPLIT
_HBPW = _HB // _NW
_HNCH = _HBPW // _CHUNK


def _gather_body(uids, iids, ut, it, uout, iout, idx_v, rows_v, sem):
    wid = lax.axis_index("s") * _NC + lax.axis_index("c")
    base = wid * _HBPW
    pltpu.sync_copy(uids.at[wid], idx_v)
    for j in range(_HNCH):
        pltpu.async_copy(ut.at[idx_v.at[j]],
                         rows_v.at[pl.ds(j * _CHUNK, _CHUNK)], sem).wait()
    pltpu.sync_copy(rows_v, uout.at[pl.ds(base, _HBPW)])
    pltpu.sync_copy(iids.at[wid], idx_v)
    for j in range(_HNCH):
        pltpu.async_copy(it.at[idx_v.at[j]],
                         rows_v.at[pl.ds(j * _CHUNK, _CHUNK)], sem).wait()
    pltpu.sync_copy(rows_v, iout.at[pl.ds(base, _HBPW)])


@functools.cache
def _gather_half():
    return pl.kernel(
        _gather_body,
        mesh=plsc.VectorSubcoreMesh(core_axis_name="c", subcore_axis_name="s"),
        out_type=[jax.ShapeDtypeStruct((_HB, _D), jnp.float32),
                  jax.ShapeDtypeStruct((_HB, _D), jnp.float32)],
        scratch_types=[pltpu.VMEM((_HNCH, _CHUNK), jnp.int32),
                       pltpu.VMEM((_HBPW, _D), jnp.float32),
                       pltpu.SemaphoreType.DMA],
    )


def _vq_body(u_ref, e_ref, cb0_ref, n0_ref, cb1_ref, n1_ref,
             logits_ref, loss_ref):
    f32 = jnp.float32
    e = e_ref[...]
    u = u_ref[...]

    def layer(r, cb_ref, n_ref):
        cb = cb_ref[...]
        rs = jnp.sum(r * r, axis=-1, keepdims=True)
        cross = lax.dot_general(r, cb, (((1,), (1,)), ((), ())),
                                preferred_element_type=f32)
        d = rs - 2.0 * cross + n_ref[...]
        m = jnp.min(d, axis=-1, keepdims=True)
        ii = lax.broadcasted_iota(jnp.int32, d.shape, 1).astype(f32)
        sel = jnp.where(d == m, ii, f32(_KP))
        amin = jnp.min(sel, axis=-1, keepdims=True)
        oh = (ii == amin).astype(f32)
        q = lax.dot_general(oh, cb, (((1,), (0,)), ((), ())),
                            preferred_element_type=f32,
                            precision=lax.Precision.HIGHEST)
        return q, m

    q0, m0 = layer(e, cb0_ref, n0_ref)
    r1 = e - q0
    q1, m1 = layer(r1, cb1_ref, n1_ref)
    t = u * (e + q0 + q1)
    g = t.shape[0] // _D
    t3 = t.reshape(g, _D, _D)
    ones = jnp.ones((g, 1, _D), f32)
    row = lax.dot_general(ones, t3, (((2,), (2,)), ((0,), (0,))),
                          preferred_element_type=f32)
    logits_ref[...] = row.reshape(g, _D)
    part = jnp.sum(m0) + jnp.sum(m1)

    @pl.when(pl.program_id(0) == 0)
    def _():
        loss_ref[0, 0] = f32(0.0)

    loss_ref[0, 0] += part


def _vq_half(u_emb, i_emb, cb0p, n0, cb1p, n1):
    bb = min(4096, _HB)
    return pl.pallas_call(
        _vq_body,
        grid=(_HB // bb,),
        in_specs=[
            pl.BlockSpec((bb, _D), lambda i: (i, 0)),
            pl.BlockSpec((bb, _D), lambda i: (i, 0)),
            pl.BlockSpec((_KP, _D), lambda i: (0, 0)),
            pl.BlockSpec((1, _KP), lambda i: (0, 0)),
            pl.BlockSpec((_KP, _D), lambda i: (0, 0)),
            pl.BlockSpec((1, _KP), lambda i: (0, 0)),
        ],
        out_specs=[
            pl.BlockSpec((bb // _D, _D), lambda i: (i, 0)),
            pl.BlockSpec((1, 1), lambda i: (0, 0),
                         memory_space=pltpu.SMEM),
        ],
        out_shape=[jax.ShapeDtypeStruct((_HB // _D, _D), jnp.float32),
                   jax.ShapeDtypeStruct((1, 1), jnp.float32)],
    )(u_emb, i_emb, cb0p, n0, cb1p, n1)


def kernel(user_ids, item_ids, labels, user_table, item_table,
           codebook0, codebook1):
    uids4 = user_ids.astype(jnp.int32).reshape(_NSPLIT, _NW, _HNCH, _CHUNK)
    iids4 = item_ids.astype(jnp.int32).reshape(_NSPLIT, _NW, _HNCH, _CHUNK)

    gathered = [_gather_half()(uids4[h], iids4[h], user_table, item_table)
                for h in range(_NSPLIT)]

    cb0p = jnp.zeros((_KP, _D), jnp.float32).at[:_K0].set(codebook0)
    cb1p = jnp.zeros((_KP, _D), jnp.float32).at[:_K1].set(codebook1)
    n0 = jnp.full((1, _KP), 1e30, jnp.float32).at[0, :_K0].set(
        jnp.sum(codebook0 * codebook0, axis=1))
    n1 = jnp.full((1, _KP), 1e30, jnp.float32).at[0, :_K1].set(
        jnp.sum(codebook1 * codebook1, axis=1))

    halves = [_vq_half(u_emb, i_emb, cb0p, n0, cb1p, n1)
              for u_emb, i_emb in gathered]

    logits = jnp.concatenate([lg for lg, _ in halves], axis=0).reshape(_B)
    loss_sum = halves[0][1][0, 0]
    for _, ls in halves[1:]:
        loss_sum = loss_sum + ls[0, 0]
    additional_loss = loss_sum * jnp.float32(1.25 / _B)
    return logits, additional_loss


# speedup vs baseline: 1.1155x; 1.1155x over previous
import functools

import jax
import jax.numpy as jnp
from jax import lax
from jax.experimental import pallas as pl
from jax.experimental.pallas import tpu as pltpu
from jax.experimental.pallas import tpu_sc as plsc

_B = 16384
_D = 128
_K0 = 100
_K1 = 10
_KP = 128
_CHUNK = 128
_NC = 2
_NS = 16
_NW = _NC * _NS
_NSPLIT = 2
_HB = _B // _NSPLIT
_HBPW = _HB // _NW
_HNCH = _HBPW // _CHUNK


def _gather_body(uids, iids, ut, it, uout, iout, idx_v, rows_v, sem):
    wid = lax.axis_index("s") * _NC + lax.axis_index("c")
    base = wid * _HBPW
    pltpu.sync_copy(uids.at[wid], idx_v)
    for j in range(_HNCH):
        pltpu.async_copy(ut.at[idx_v.at[j]],
                         rows_v.at[pl.ds(j * _CHUNK, _CHUNK)], sem).wait()
    pltpu.sync_copy(rows_v, uout.at[pl.ds(base, _HBPW)])
    pltpu.sync_copy(iids.at[wid], idx_v)
    for j in range(_HNCH):
        pltpu.async_copy(it.at[idx_v.at[j]],
                         rows_v.at[pl.ds(j * _CHUNK, _CHUNK)], sem).wait()
    pltpu.sync_copy(rows_v, iout.at[pl.ds(base, _HBPW)])


@functools.cache
def _gather_half():
    return pl.kernel(
        _gather_body,
        mesh=plsc.VectorSubcoreMesh(core_axis_name="c", subcore_axis_name="s"),
        out_type=[jax.ShapeDtypeStruct((_HB, _D), jnp.float32),
                  jax.ShapeDtypeStruct((_HB, _D), jnp.float32)],
        scratch_types=[pltpu.VMEM((_HNCH, _CHUNK), jnp.int32),
                       pltpu.VMEM((_HBPW, _D), jnp.float32),
                       pltpu.SemaphoreType.DMA],
    )


def _vq_body(u_ref, e_ref, cb0_ref, n0_ref, cb1_ref, n1_ref,
             logits_ref, loss_ref):
    f32 = jnp.float32
    e = e_ref[...]
    u = u_ref[...]

    def layer(r, cb_ref, n_ref):
        cb = cb_ref[...]
        rs = jnp.sum(r * r, axis=-1, keepdims=True)
        cross = lax.dot_general(r, cb, (((1,), (1,)), ((), ())),
                                preferred_element_type=f32)
        d = rs - 2.0 * cross + n_ref[...]
        m = jnp.min(d, axis=-1, keepdims=True)
        ii = lax.broadcasted_iota(jnp.int32, d.shape, 1).astype(f32)
        sel = jnp.where(d == m, ii, f32(_KP))
        amin = jnp.min(sel, axis=-1, keepdims=True)
        oh = (ii == amin).astype(f32)
        q = lax.dot_general(oh, cb, (((1,), (0,)), ((), ())),
                            preferred_element_type=f32,
                            precision=lax.Precision.HIGHEST)
        return q, m

    q0, m0 = layer(e, cb0_ref, n0_ref)
    r1 = e - q0
    q1, m1 = layer(r1, cb1_ref, n1_ref)
    t = u * (e + q0 + q1)
    g = t.shape[0] // _D
    t3 = t.reshape(g, _D, _D)
    ones = jnp.ones((g, 1, _D), f32)
    row = lax.dot_general(ones, t3, (((2,), (2,)), ((0,), (0,))),
                          preferred_element_type=f32)
    logits_ref[...] = row.reshape(g, _D)
    part = jnp.sum(m0) + jnp.sum(m1)

    @pl.when(pl.program_id(0) == 0)
    def _():
        loss_ref[0, 0] = f32(0.0)

    loss_ref[0, 0] += part


def _vq_half(u_emb, i_emb, cb0p, n0, cb1p, n1):
    bb = min(4096, _HB)
    return pl.pallas_call(
        _vq_body,
        grid=(_HB // bb,),
        in_specs=[
            pl.BlockSpec((bb, _D), lambda i: (i, 0)),
            pl.BlockSpec((bb, _D), lambda i: (i, 0)),
            pl.BlockSpec((_KP, _D), lambda i: (0, 0)),
            pl.BlockSpec((1, _KP), lambda i: (0, 0)),
            pl.BlockSpec((_KP, _D), lambda i: (0, 0)),
            pl.BlockSpec((1, _KP), lambda i: (0, 0)),
        ],
        out_specs=[
            pl.BlockSpec((bb // _D, _D), lambda i: (i, 0)),
            pl.BlockSpec((1, 1), lambda i: (0, 0),
                         memory_space=pltpu.SMEM),
        ],
        out_shape=[jax.ShapeDtypeStruct((_HB // _D, _D), jnp.float32),
                   jax.ShapeDtypeStruct((1, 1), jnp.float32)],
    )(u_emb, i_emb, cb0p, n0, cb1p, n1)


def kernel(user_ids, item_ids, labels, user_table, item_table,
           codebook0, codebook1):
    uids4 = user_ids.astype(jnp.int32).reshape(_NSPLIT, _NW, _HNCH, _CHUNK)
    iids4 = item_ids.astype(jnp.int32).reshape(_NSPLIT, _NW, _HNCH, _CHUNK)

    gathered = [_gather_half()(uids4[h], iids4[h], user_table, item_table)
                for h in range(_NSPLIT)]

    cb0p = jnp.zeros((_KP, _D), jnp.float32).at[:_K0].set(codebook0)
    cb1p = jnp.zeros((_KP, _D), jnp.float32).at[:_K1].set(codebook1)
    n0 = jnp.full((1, _KP), 1e30, jnp.float32).at[0, :_K0].set(
        jnp.sum(codebook0 * codebook0, axis=1))
    n1 = jnp.full((1, _KP), 1e30, jnp.float32).at[0, :_K1].set(
        jnp.sum(codebook1 * codebook1, axis=1))

    halves = [_vq_half(u_emb, i_emb, cb0p, n0, cb1p, n1)
              for u_emb, i_emb in gathered]

    logits = jnp.concatenate([lg for lg, _ in halves], axis=0).reshape(_B)
    loss_sum = halves[0][1][0, 0]
    for _, ls in halves[1:]:
        loss_sum = loss_sum + ls[0, 0]
    additional_loss = loss_sum * jnp.float32(1.25 / _B)
    return logits, additional_loss


# speedup vs baseline: 1.3074x; 1.1720x over previous
import functools

import jax
import jax.numpy as jnp
from jax import lax
from jax.experimental import pallas as pl
from jax.experimental.pallas import tpu as pltpu
from jax.experimental.pallas import tpu_sc as plsc

_B = 16384
_D = 128
_K0 = 100
_K1 = 10
_KP = 128
_CHUNK = 128
_NC = 2
_NS = 16
_NW = _NC * _NS
_NSPLIT = 2
_HB = _B // _NSPLIT
_HBPW = _HB // _NW
_HNCH = _HBPW // _CHUNK


def _gather_body(uids, iids, ut, it, uout, iout, idx_v, rows_v, sem):
    wid = lax.axis_index("s") * _NC + lax.axis_index("c")
    base = wid * _HBPW
    pltpu.sync_copy(uids.at[wid], idx_v)
    for j in range(_HNCH):
        pltpu.async_copy(ut.at[idx_v.at[j]],
                         rows_v.at[pl.ds(j * _CHUNK, _CHUNK)], sem).wait()
    pltpu.sync_copy(rows_v, uout.at[pl.ds(base, _HBPW)])
    pltpu.sync_copy(iids.at[wid], idx_v)
    for j in range(_HNCH):
        pltpu.async_copy(it.at[idx_v.at[j]],
                         rows_v.at[pl.ds(j * _CHUNK, _CHUNK)], sem).wait()
    pltpu.sync_copy(rows_v, iout.at[pl.ds(base, _HBPW)])


@functools.cache
def _gather_half():
    return pl.kernel(
        _gather_body,
        mesh=plsc.VectorSubcoreMesh(core_axis_name="c", subcore_axis_name="s"),
        out_type=[jax.ShapeDtypeStruct((_HB, _D), jnp.float32),
                  jax.ShapeDtypeStruct((_HB, _D), jnp.float32)],
        scratch_types=[pltpu.VMEM((_HNCH, _CHUNK), jnp.int32),
                       pltpu.VMEM((_HBPW, _D), jnp.float32),
                       pltpu.SemaphoreType.DMA],
    )


def _vq_body(u_ref, e_ref, cb0_ref, n0_ref, a0_ref, b0_ref, c0_ref,
             cb1_ref, n1_ref, a1_ref, b1_ref, c1_ref,
             logits_ref, loss_ref):
    f32 = jnp.float32
    bf16 = jnp.bfloat16
    e = e_ref[...]
    u = u_ref[...]

    def layer(r, cb_ref, n_ref, a_ref, b_ref, c_ref):
        cb = cb_ref[...]
        rs = jnp.sum(r * r, axis=-1, keepdims=True)
        cross = lax.dot_general(r, cb, (((1,), (1,)), ((), ())),
                                preferred_element_type=f32)
        d = rs - 2.0 * cross + n_ref[...]
        m = jnp.min(d, axis=-1, keepdims=True)
        ii = lax.broadcasted_iota(jnp.int32, d.shape, 1).astype(f32)
        sel = jnp.where(d == m, ii, f32(_KP))
        amin = jnp.min(sel, axis=-1, keepdims=True)
        ohb = (ii == amin).astype(bf16)
        def sel_dot(p_ref):
            return lax.dot_general(ohb, p_ref[...], (((1,), (0,)), ((), ())),
                                   preferred_element_type=f32)
        q = sel_dot(a_ref) + (sel_dot(b_ref) + sel_dot(c_ref))
        return q, m

    q0, m0 = layer(e, cb0_ref, n0_ref, a0_ref, b0_ref, c0_ref)
    r1 = e - q0
    q1, m1 = layer(r1, cb1_ref, n1_ref, a1_ref, b1_ref, c1_ref)
    t = u * (e + q0 + q1)
    g = t.shape[0] // _D
    t3 = t.reshape(g, _D, _D)
    ones = jnp.ones((g, 1, _D), f32)
    row = lax.dot_general(ones, t3, (((2,), (2,)), ((0,), (0,))),
                          preferred_element_type=f32)
    logits_ref[...] = row.reshape(g, _D)
    mm3 = (m0 + m1).reshape(g, _D, 1)
    psum = lax.dot_general(ones, mm3, (((2,), (1,)), ((0,), (0,))),
                           preferred_element_type=f32)
    part = jnp.sum(psum)

    @pl.when(pl.program_id(0) == 0)
    def _():
        loss_ref[0, 0] = f32(0.0)

    loss_ref[0, 0] += part


def _vq_half(u_emb, i_emb, cb0p, n0, s0, cb1p, n1, s1):
    bb = min(4096, _HB)
    cbspec = [pl.BlockSpec((_KP, _D), lambda i: (0, 0)),
              pl.BlockSpec((1, _KP), lambda i: (0, 0)),
              pl.BlockSpec((_KP, _D), lambda i: (0, 0)),
              pl.BlockSpec((_KP, _D), lambda i: (0, 0)),
              pl.BlockSpec((_KP, _D), lambda i: (0, 0))]
    return pl.pallas_call(
        _vq_body,
        grid=(_HB // bb,),
        in_specs=[
            pl.BlockSpec((bb, _D), lambda i: (i, 0)),
            pl.BlockSpec((bb, _D), lambda i: (i, 0)),
        ] + cbspec + cbspec,
        out_specs=[
            pl.BlockSpec((bb // _D, _D), lambda i: (i, 0)),
            pl.BlockSpec((1, 1), lambda i: (0, 0),
                         memory_space=pltpu.SMEM),
        ],
        out_shape=[jax.ShapeDtypeStruct((_HB // _D, _D), jnp.float32),
                   jax.ShapeDtypeStruct((1, 1), jnp.float32)],
    )(u_emb, i_emb, cb0p, n0, *s0, cb1p, n1, *s1)


def kernel(user_ids, item_ids, labels, user_table, item_table,
           codebook0, codebook1):
    uids4 = user_ids.astype(jnp.int32).reshape(_NSPLIT, _NW, _HNCH, _CHUNK)
    iids4 = item_ids.astype(jnp.int32).reshape(_NSPLIT, _NW, _HNCH, _CHUNK)

    gathered = [_gather_half()(uids4[h], iids4[h], user_table, item_table)
                for h in range(_NSPLIT)]

    cb0p = jnp.zeros((_KP, _D), jnp.float32).at[:_K0].set(codebook0)
    cb1p = jnp.zeros((_KP, _D), jnp.float32).at[:_K1].set(codebook1)
    n0 = jnp.full((1, _KP), 1e30, jnp.float32).at[0, :_K0].set(
        jnp.sum(codebook0 * codebook0, axis=1))
    n1 = jnp.full((1, _KP), 1e30, jnp.float32).at[0, :_K1].set(
        jnp.sum(codebook1 * codebook1, axis=1))

    def split3(cb):
        a = cb.astype(jnp.bfloat16)
        r = cb - a.astype(jnp.float32)
        b = r.astype(jnp.bfloat16)
        c = (r - b.astype(jnp.float32)).astype(jnp.bfloat16)
        return a, b, c

    s0 = split3(cb0p)
    s1 = split3(cb1p)

    halves = [_vq_half(u_emb, i_emb, cb0p, n0, s0, cb1p, n1, s1)
              for u_emb, i_emb in gathered]

    logits = jnp.concatenate([lg for lg, _ in halves], axis=0).reshape(_B)
    loss_sum = halves[0][1][0, 0]
    for _, ls in halves[1:]:
        loss_sum = loss_sum + ls[0, 0]
    additional_loss = loss_sum * jnp.float32(1.25 / _B)
    return logits, additional_loss
